# Initial kernel scaffold; baseline (speedup 1.0000x reference)
import jax
import jax.numpy as jnp
from jax.experimental import pallas as pl


def kernel(queries, W, corpus_keys, corpus_doc_ids):
    raise NotImplementedError("write your pallas kernel here")


# speedup vs baseline: 1.3520x; 1.3520x over previous
import jax
import jax.numpy as jnp
from jax.experimental import pallas as pl
from jax.experimental.pallas import tpu as pltpu

_N_DOCS = 100000
_K = 16
_QT = 256
_CT = 2048
_N_PAD = 100352


def _topk_body(q_ref, w_ref, k_ref, ids_out, vals_out, proj_ref, topv_ref, topi_ref):
    ci = pl.program_id(1)
    nc = pl.num_programs(1)

    @pl.when(ci == 0)
    def _init():
        proj_ref[...] = jnp.dot(q_ref[...], w_ref[...].T,
                                preferred_element_type=jnp.float32)
        topv_ref[...] = jnp.full((_QT, _K), -jnp.inf, jnp.float32)
        topi_ref[...] = jnp.zeros((_QT, _K), jnp.int32)

    scores = jax.lax.dot_general(
        proj_ref[...], k_ref[...],
        (((1,), (1,)), ((), ())),
        preferred_element_type=jnp.float32)

    col = jax.lax.broadcasted_iota(jnp.int32, (_QT, _CT), 1) + ci * _CT
    scores = jnp.where(col < _N_DOCS, scores, -jnp.inf)

    arr = jnp.concatenate([topv_ref[...], scores], axis=1)
    idxarr = jnp.concatenate([topi_ref[...], col], axis=1)
    width = _K + _CT
    iota = jax.lax.broadcasted_iota(jnp.int32, (_QT, width), 1)

    newv = []
    newi = []
    for _ in range(_K):
        m = jnp.max(arr, axis=1, keepdims=True)
        cand = jnp.where(arr == m, iota, jnp.int32(2**30))
        arg = jnp.min(cand, axis=1, keepdims=True)
        doc = jnp.min(jnp.where(iota == arg, idxarr, jnp.int32(2**30)),
                      axis=1, keepdims=True)
        newv.append(m)
        newi.append(doc)
        arr = jnp.where(iota == arg, -jnp.inf, arr)

    topv_ref[...] = jnp.concatenate(newv, axis=1)
    topi_ref[...] = jnp.concatenate(newi, axis=1)

    @pl.when(ci == nc - 1)
    def _emit():
        vals_out[...] = topv_ref[...]
        ids_out[...] = topi_ref[...]


def kernel(queries, W, corpus_keys, corpus_doc_ids):
    nq = queries.shape[0]
    keys_pad = jnp.pad(corpus_keys, ((0, _N_PAD - _N_DOCS), (0, 0)))

    grid = (nq // _QT, _N_PAD // _CT)
    ids, vals = pl.pallas_call(
        _topk_body,
        grid=grid,
        in_specs=[
            pl.BlockSpec((_QT, 128), lambda i, j: (i, 0)),
            pl.BlockSpec((128, 128), lambda i, j: (0, 0)),
            pl.BlockSpec((_CT, 128), lambda i, j: (j, 0)),
        ],
        out_specs=[
            pl.BlockSpec((_QT, _K), lambda i, j: (i, 0)),
            pl.BlockSpec((_QT, _K), lambda i, j: (i, 0)),
        ],
        out_shape=[
            jax.ShapeDtypeStruct((nq, _K), jnp.int32),
            jax.ShapeDtypeStruct((nq, _K), jnp.float32),
        ],
        scratch_shapes=[
            pltpu.VMEM((_QT, 128), jnp.float32),
            pltpu.VMEM((_QT, _K), jnp.float32),
            pltpu.VMEM((_QT, _K), jnp.int32),
        ],
        compiler_params=pltpu.CompilerParams(
            dimension_semantics=("arbitrary", "arbitrary")),
    )(queries.astype(jnp.float32), W, keys_pad)

    top_doc_ids = jnp.take(corpus_doc_ids, ids, axis=0)
    return top_doc_ids, vals
